# baseline (speedup 1.0000x reference)
import jax
import jax.numpy as jnp
from jax.experimental import pallas as pl
from jax.experimental.pallas import tpu as pltpu

_E = 8
_D = 2048
_H = 2048
_B = 2048
_K_TOP = 32

_DT = 256
_NC = _D // _DT
_BT = 256
_NM = _B // _BT


def _gate_w(gw, gb):
    a_mean = (_D + _K_TOP) / float(_D)
    t_mean = 1.0
    logits = t_mean * gw[0:1, :] + a_mean * gw[1:2, :] + gb
    m = jnp.max(logits)
    p = jnp.exp(logits - m)
    return p / jnp.sum(p)


def _body(gw_ref, gb_ref, experts_ref, x_ref, out_ref, cmb_ref, xn_ref):
    j = pl.program_id(0)

    @pl.when(j < _NC)
    def _combine():
        w = _gate_w(gw_ref[...], gb_ref[...])
        blk = experts_ref[...]
        acc = blk[0] * w[0, 0]
        for e in range(1, _E):
            acc = acc + blk[e] * w[0, e]
        cmb_ref[pl.ds(j * _DT, _DT), :] = acc.astype(jnp.bfloat16)
        x = x_ref[...]
        mean = jnp.mean(x, axis=-1, keepdims=True)
        cen = x - mean
        std = jnp.sqrt(jnp.mean(cen * cen, axis=-1, keepdims=True))
        xn_ref[pl.ds(j * _BT, _BT), :] = (
            cen / (std + 1e-6)).astype(jnp.bfloat16)

    @pl.when(j >= _NC)
    def _matmul():
        bt = j - _NC
        out_ref[...] = jnp.dot(xn_ref[pl.ds(bt * _BT, _BT), :], cmb_ref[...],
                               preferred_element_type=jnp.float32)


def _e_idx(j):
    return (0, jnp.minimum(j, _NC - 1), 0)


def _x_idx(j):
    return (jnp.minimum(j, _NM - 1), 0)


def _o_idx(j):
    return (jnp.clip(j - _NC, 0, _NM - 1), 0)


def kernel(query_embedding, gate_W, gate_b, experts):
    gb2 = gate_b.reshape(1, _E)

    out = pl.pallas_call(
        _body,
        grid=(_NC + _NM,),
        in_specs=[
            pl.BlockSpec((12, _E), lambda j: (0, 0)),
            pl.BlockSpec((1, _E), lambda j: (0, 0)),
            pl.BlockSpec((_E, _DT, _H), _e_idx),
            pl.BlockSpec((_BT, _D), _x_idx),
        ],
        out_specs=pl.BlockSpec((_BT, _H), _o_idx),
        out_shape=jax.ShapeDtypeStruct((_B, _H), jnp.float32),
        scratch_shapes=[
            pltpu.VMEM((_D, _H), jnp.bfloat16),
            pltpu.VMEM((_B, _D), jnp.bfloat16),
        ],
        compiler_params=pltpu.CompilerParams(
            vmem_limit_bytes=64 * 1024 * 1024),
    )(gate_W, gb2, experts, query_embedding)

    return out


# speedup vs baseline: 1.0255x; 1.0255x over previous
import jax
import jax.numpy as jnp
from jax.experimental import pallas as pl
from jax.experimental.pallas import tpu as pltpu

_E = 8
_D = 2048
_H = 2048
_B = 2048
_K_TOP = 32

_DT = 256
_NC = _D // _DT
_BT = 256
_NM = _B // _BT


def _gate_w(gw, gb):
    a_mean = (_D + _K_TOP) / float(_D)
    t_mean = 1.0
    logits = t_mean * gw[0:1, :] + a_mean * gw[1:2, :] + gb
    m = jnp.max(logits)
    p = jnp.exp(logits - m)
    return p / jnp.sum(p)


def _body(gw_ref, gb_ref, experts_ref, x_ref, out_ref, cmb_ref):
    j = pl.program_id(0)

    @pl.when(j < _NC)
    def _combine():
        w = _gate_w(gw_ref[...], gb_ref[...])
        blk = experts_ref[...]
        acc = blk[0] * w[0, 0]
        for e in range(1, _E):
            acc = acc + blk[e] * w[0, e]
        cmb_ref[pl.ds(j * _DT, _DT), :] = acc.astype(jnp.bfloat16)

    @pl.when(j >= _NC)
    def _matmul():
        x = x_ref[...]
        mean = jnp.mean(x, axis=-1, keepdims=True)
        cen = x - mean
        std = jnp.sqrt(jnp.mean(cen * cen, axis=-1, keepdims=True))
        xn = (cen / (std + 1e-6)).astype(jnp.bfloat16)
        out_ref[...] = jnp.dot(xn, cmb_ref[...],
                               preferred_element_type=jnp.float32)


def _e_idx(j):
    return (0, jnp.minimum(j, _NC - 1), 0)


def _x_idx(j):
    return (jnp.clip(j - _NC, 0, _NM - 1), 0)


def kernel(query_embedding, gate_W, gate_b, experts):
    gb2 = gate_b.reshape(1, _E)

    out = pl.pallas_call(
        _body,
        grid=(_NC + _NM,),
        in_specs=[
            pl.BlockSpec((12, _E), lambda j: (0, 0)),
            pl.BlockSpec((1, _E), lambda j: (0, 0)),
            pl.BlockSpec((_E, _DT, _H), _e_idx),
            pl.BlockSpec((_BT, _D), _x_idx),
        ],
        out_specs=pl.BlockSpec((_BT, _H), _x_idx),
        out_shape=jax.ShapeDtypeStruct((_B, _H), jnp.float32),
        scratch_shapes=[
            pltpu.VMEM((_D, _H), jnp.bfloat16),
        ],
    )(gate_W, gb2, experts, query_embedding)

    return out
